# baseline (speedup 1.0000x reference)
import functools

import numpy as np
import jax
import jax.numpy as jnp
from jax import lax
from jax.experimental import pallas as pl
from jax.experimental.pallas import tpu as pltpu
from jax.experimental.pallas import tpu_sc as plsc

_T, _B, _C = 1024, 64, 192
_RATIO = 0.75
_REMAIN = int(_T * (1.0 - _RATIO))
_ROWS = _REMAIN * _B
_NC, _NS = 2, 16
_NW = _NC * _NS
_RPW = _ROWS // _NW
_CHUNK = 128
_NCHUNK = _RPW // _CHUNK

_cache = {}


def _rotl(x, r):
    return ((x << np.uint32(r)) | (x >> np.uint32(32 - r))).astype(np.uint32)


def _threefry2x32(k1, k2, x0, x1):
    x0 = x0.astype(np.uint32).copy()
    x1 = x1.astype(np.uint32).copy()
    ks0, ks1 = np.uint32(k1), np.uint32(k2)
    ks2 = np.uint32(ks0 ^ ks1 ^ np.uint32(0x1BD11BDA))
    rot1, rot2 = (13, 15, 26, 6), (17, 29, 16, 24)
    x0 = (x0 + ks0).astype(np.uint32)
    x1 = (x1 + ks1).astype(np.uint32)
    inject = [(ks1, ks2, 1), (ks2, ks0, 2), (ks0, ks1, 3),
              (ks1, ks2, 4), (ks2, ks0, 5)]
    for i, rots in enumerate((rot1, rot2, rot1, rot2, rot1)):
        for r in rots:
            x0 = (x0 + x1).astype(np.uint32)
            x1 = _rotl(x1, r)
            x1 = (x1 ^ x0).astype(np.uint32)
        a, b, c = inject[i]
        x0 = (x0 + a).astype(np.uint32)
        x1 = (x1 + b + np.uint32(c)).astype(np.uint32)
    return x0, x1


def _split(key, num):
    b1, b2 = _threefry2x32(key[0], key[1],
                           np.zeros(num, dtype=np.uint32),
                           np.arange(num, dtype=np.uint32))
    return np.stack([b1, b2], axis=1)


def _random_bits32(key, n):
    b1, b2 = _threefry2x32(key[0], key[1],
                           np.zeros(n, dtype=np.uint32),
                           np.arange(n, dtype=np.uint32))
    return (b1 ^ b2).astype(np.uint32)


def _permutation_arange(key, n):
    num_rounds = int(np.ceil(3 * np.log(max(1, n)) / np.log(2**32 - 1)))
    x = np.arange(n, dtype=np.int32)
    for _ in range(num_rounds):
        ks = _split(key, 2)
        key, subkey = ks[0], ks[1]
        order = np.argsort(_random_bits32(subkey, n), kind="stable")
        x = x[order]
    return x


def _indices():
    if "fwd" not in _cache:
        keys = _split(np.array([0, 42], dtype=np.uint32), _B)
        fwd = np.stack(
            [_permutation_arange(keys[i], _T) for i in range(_B)]
        ).T.astype(np.int32)
        bwd = np.argsort(fwd, axis=0).astype(np.int32)
        flat = (fwd[:_REMAIN].astype(np.int64) * _B
                + np.arange(_B, dtype=np.int64)[None, :])
        flat = flat.reshape(_NW, _NCHUNK, _CHUNK).astype(np.int32)
        _cache["fwd"], _cache["bwd"], _cache["flat"] = fwd, bwd, flat
    return _cache["fwd"], _cache["bwd"], _cache["flat"]


def _gather_body(src, idx, out, idx_v, rows_v, sem):
    wid = lax.axis_index("s") * _NC + lax.axis_index("c")
    pltpu.sync_copy(idx.at[wid], idx_v)
    base = wid * _RPW
    for j in range(_NCHUNK):
        pltpu.async_copy(src.at[idx_v.at[j]], rows_v, sem).wait()
        pltpu.sync_copy(rows_v, out.at[pl.ds(base + j * _CHUNK, _CHUNK)])


def _build_gather():
    if "gather" not in _cache:
        _cache["gather"] = pl.kernel(
            _gather_body,
            out_type=jax.ShapeDtypeStruct((_ROWS, _C), jnp.float32),
            mesh=plsc.VectorSubcoreMesh(core_axis_name="c",
                                        subcore_axis_name="s",
                                        num_cores=_NC, num_subcores=_NS),
            scratch_types=[
                pltpu.VMEM((_NCHUNK, _CHUNK), jnp.int32),
                pltpu.VMEM((_CHUNK, _C), jnp.float32),
                pltpu.SemaphoreType.DMA,
            ],
            compiler_params=pltpu.CompilerParams(use_tc_tiling_on_sc=False),
        )
    return _cache["gather"]


def kernel(patches):
    fwd, bwd, flat = _indices()
    src = patches.reshape(_T * _B, _C)
    out = _build_gather()(src, jnp.asarray(flat))
    return (out.reshape(_REMAIN, _B, _C),
            jnp.asarray(fwd), jnp.asarray(bwd))
